# baseline (speedup 1.0000x reference)
import functools

import jax
import jax.numpy as jnp
from jax.experimental import pallas as pl
from jax.experimental.pallas import tpu as pltpu


def _conv_body(x_ref, w_ref, b_ref, o_ref, xz_ref, *, H, W, rc, taps):
    c = pl.program_id(1)
    Cin = x_ref.shape[3]
    Cout = o_ref.shape[2]
    L = rc * W

    @pl.when(c == 0)
    def _fill():
        xb = x_ref[0].astype(jnp.bfloat16)
        for k, (dy, dx) in enumerate(taps):
            ch = slice(k * Cin, (k + 1) * Cin)
            h0, h1 = max(0, -dy), min(H, H - dy)
            w0, w1 = max(0, -dx), min(W, W - dx)
            xz_ref[h0:h1, w0:w1, ch] = xb[h0 + dy:h1 + dy, w0 + dx:w1 + dx, :]
            if h0 > 0:
                xz_ref[0:h0, :, ch] = jnp.zeros((h0, W, Cin), jnp.bfloat16)
            if h1 < H:
                xz_ref[h1:H, :, ch] = jnp.zeros((H - h1, W, Cin), jnp.bfloat16)
            if w0 > 0:
                xz_ref[h0:h1, 0:w0, ch] = jnp.zeros((h1 - h0, w0, Cin),
                                                    jnp.bfloat16)
            if w1 < W:
                xz_ref[h0:h1, w1:W, ch] = jnp.zeros((h1 - h0, W - w1, Cin),
                                                    jnp.bfloat16)

    row0 = c * rc
    tap = xz_ref[pl.ds(row0, rc), :, :].reshape(L, len(taps) * Cin)
    acc = jnp.broadcast_to(b_ref[...], (L, Cout))
    acc = acc + jnp.dot(tap, w_ref[...], preferred_element_type=jnp.float32)
    o_ref[0] = acc


def _pick_chunk_rows(H, W, budget=512):
    for rc in range(min(H, budget // W), 0, -1):
        if H % rc == 0:
            return rc
    return 1


def kernel(x, weight, bias):
    N, Cin, H, W = x.shape
    Cout, Cin_w, kh, kw = weight.shape
    assert Cin_w == Cin and kh == 3 and kw == 3

    HW = H * W
    rc = _pick_chunk_rows(H, W)
    L = rc * W

    xh = jnp.transpose(x, (0, 2, 3, 1))
    wt = (jnp.transpose(weight, (2, 3, 1, 0))
          .reshape(kh * kw * Cin, Cout).astype(jnp.bfloat16))
    b2 = bias.astype(jnp.float32).reshape(1, Cout)

    taps = [(dy, dx) for dy in (-1, 0, 1) for dx in (-1, 0, 1)]
    body = functools.partial(_conv_body, H=H, W=W, rc=rc, taps=taps)

    grid = (N, H // rc)
    out = pl.pallas_call(
        body,
        out_shape=jax.ShapeDtypeStruct((N, HW, Cout), jnp.float32),
        grid=grid,
        in_specs=[
            pl.BlockSpec((1, H, W, Cin), lambda n, c: (n, 0, 0, 0)),
            pl.BlockSpec((kh * kw * Cin, Cout), lambda n, c: (0, 0)),
            pl.BlockSpec((1, Cout), lambda n, c: (0, 0)),
        ],
        out_specs=pl.BlockSpec((1, L, Cout), lambda n, c: (n, c, 0)),
        scratch_shapes=[pltpu.VMEM((H, W, kh * kw * Cin), jnp.bfloat16)],
        compiler_params=pltpu.CompilerParams(
            dimension_semantics=("parallel", "arbitrary"),
            vmem_limit_bytes=48 * 1024 * 1024),
    )(xh, wt, b2)

    return jnp.transpose(out.reshape(N, H, W, Cout), (0, 3, 1, 2))


# speedup vs baseline: 2.2192x; 2.2192x over previous
import functools

import jax
import jax.numpy as jnp
from jax.experimental import pallas as pl
from jax.experimental.pallas import tpu as pltpu


def _conv_body(x_hbm, w_ref, b_ref, o_ref, xraw_ref, xz_ref, sem,
               *, N, H, W, rc, taps):
    n = pl.program_id(0)
    Cin = x_hbm.shape[3]
    Cout = o_ref.shape[2]
    L = rc * W
    slot = jax.lax.rem(n, 2)

    @pl.when(n == 0)
    def _first():
        pltpu.make_async_copy(x_hbm.at[n], xraw_ref.at[slot], sem.at[slot]
                              ).start()
    pltpu.make_async_copy(x_hbm.at[n], xraw_ref.at[slot], sem.at[slot]).wait()

    @pl.when(n + 1 < N)
    def _prefetch():
        nxt = jax.lax.rem(n + 1, 2)
        pltpu.make_async_copy(x_hbm.at[n + 1], xraw_ref.at[nxt], sem.at[nxt]
                              ).start()

    xb = xraw_ref[slot].astype(jnp.bfloat16)
    for k, (dy, dx) in enumerate(taps):
        ch = slice(k * Cin, (k + 1) * Cin)
        h0, h1 = max(0, -dy), min(H, H - dy)
        w0, w1 = max(0, -dx), min(W, W - dx)
        xz_ref[h0:h1, w0:w1, ch] = xb[h0 + dy:h1 + dy, w0 + dx:w1 + dx, :]
        if h0 > 0:
            xz_ref[0:h0, :, ch] = jnp.zeros((h0, W, Cin), jnp.bfloat16)
        if h1 < H:
            xz_ref[h1:H, :, ch] = jnp.zeros((H - h1, W, Cin), jnp.bfloat16)
        if w0 > 0:
            xz_ref[h0:h1, 0:w0, ch] = jnp.zeros((h1 - h0, w0, Cin),
                                                jnp.bfloat16)
        if w1 < W:
            xz_ref[h0:h1, w1:W, ch] = jnp.zeros((h1 - h0, W - w1, Cin),
                                                jnp.bfloat16)

    KT = len(taps) * Cin
    for c in range(H // rc):
        tap = xz_ref[c * rc:(c + 1) * rc, :, :].reshape(L, KT)
        acc = jnp.broadcast_to(b_ref[...], (L, Cout))
        acc = acc + jnp.dot(tap, w_ref[...],
                            preferred_element_type=jnp.float32)
        o_ref[0, c * L:(c + 1) * L, :] = acc


def _pick_chunk_rows(H, W, budget=512):
    for rc in range(min(H, budget // W), 0, -1):
        if H % rc == 0:
            return rc
    return 1


def kernel(x, weight, bias):
    N, Cin, H, W = x.shape
    Cout, Cin_w, kh, kw = weight.shape
    assert Cin_w == Cin and kh == 3 and kw == 3

    HW = H * W
    rc = _pick_chunk_rows(H, W)

    xh = jnp.transpose(x, (0, 2, 3, 1))
    wt = (jnp.transpose(weight, (2, 3, 1, 0))
          .reshape(kh * kw * Cin, Cout).astype(jnp.bfloat16))
    b2 = bias.astype(jnp.float32).reshape(1, Cout)

    taps = [(dy, dx) for dy in (-1, 0, 1) for dx in (-1, 0, 1)]
    body = functools.partial(_conv_body, N=N, H=H, W=W, rc=rc, taps=taps)

    out = pl.pallas_call(
        body,
        out_shape=jax.ShapeDtypeStruct((N, HW, Cout), jnp.float32),
        grid=(N,),
        in_specs=[
            pl.BlockSpec(memory_space=pl.ANY),
            pl.BlockSpec((kh * kw * Cin, Cout), lambda n: (0, 0)),
            pl.BlockSpec((1, Cout), lambda n: (0, 0)),
        ],
        out_specs=pl.BlockSpec((1, HW, Cout), lambda n: (n, 0, 0)),
        scratch_shapes=[pltpu.VMEM((2, H, W, Cin), x.dtype),
                        pltpu.VMEM((H, W, kh * kw * Cin), jnp.bfloat16),
                        pltpu.SemaphoreType.DMA((2,))],
        compiler_params=pltpu.CompilerParams(
            dimension_semantics=("arbitrary",),
            vmem_limit_bytes=48 * 1024 * 1024),
    )(xh, wt, b2)

    return jnp.transpose(out.reshape(N, H, W, Cout), (0, 3, 1, 2))


# speedup vs baseline: 2.2212x; 1.0009x over previous
import functools

import jax
import jax.numpy as jnp
from jax.experimental import pallas as pl
from jax.experimental.pallas import tpu as pltpu


def _conv_body(x_hbm, w_ref, b_ref, o_ref, xraw_ref, xz_ref, sem,
               *, N, H, W, rc, taps):
    n = pl.program_id(0)
    Cin = x_hbm.shape[3]
    Cout = o_ref.shape[2]
    L = rc * W
    slot = jax.lax.rem(n, 2)

    @pl.when(n == 0)
    def _first():
        pltpu.make_async_copy(x_hbm.at[n], xraw_ref.at[slot], sem.at[slot]
                              ).start()
    pltpu.make_async_copy(x_hbm.at[n], xraw_ref.at[slot], sem.at[slot]).wait()

    @pl.when(n + 1 < N)
    def _prefetch():
        nxt = jax.lax.rem(n + 1, 2)
        pltpu.make_async_copy(x_hbm.at[n + 1], xraw_ref.at[nxt], sem.at[nxt]
                              ).start()

    xb = xraw_ref[slot].astype(jnp.bfloat16)
    for k, (dy, dx) in enumerate(taps):
        ch = slice(k * Cin, (k + 1) * Cin)
        h0, h1 = max(0, -dy), min(H, H - dy)
        w0, w1 = max(0, -dx), min(W, W - dx)
        xz_ref[h0:h1, w0:w1, ch] = xb[h0 + dy:h1 + dy, w0 + dx:w1 + dx, :]
        if h0 > 0:
            xz_ref[0:h0, :, ch] = jnp.zeros((h0, W, Cin), jnp.bfloat16)
        if h1 < H:
            xz_ref[h1:H, :, ch] = jnp.zeros((H - h1, W, Cin), jnp.bfloat16)
        if w0 > 0:
            xz_ref[h0:h1, 0:w0, ch] = jnp.zeros((h1 - h0, w0, Cin),
                                                jnp.bfloat16)
        if w1 < W:
            xz_ref[h0:h1, w1:W, ch] = jnp.zeros((h1 - h0, W - w1, Cin),
                                                jnp.bfloat16)

    KT = len(taps) * Cin
    for c in range(H // rc):
        tap = xz_ref[c * rc:(c + 1) * rc, :, :].reshape(L, KT)
        acc = jnp.broadcast_to(b_ref[...], (L, Cout))
        acc = acc + jnp.dot(tap, w_ref[...],
                            preferred_element_type=jnp.float32)
        o_ref[0, c * L:(c + 1) * L, :] = acc


def _pick_chunk_rows(H, W, budget=1024):
    for rc in range(min(H, budget // W), 0, -1):
        if H % rc == 0:
            return rc
    return 1


def kernel(x, weight, bias):
    N, Cin, H, W = x.shape
    Cout, Cin_w, kh, kw = weight.shape
    assert Cin_w == Cin and kh == 3 and kw == 3

    HW = H * W
    rc = _pick_chunk_rows(H, W)

    xh = jnp.transpose(x, (0, 2, 3, 1))
    wt = (jnp.transpose(weight, (2, 3, 1, 0))
          .reshape(kh * kw * Cin, Cout).astype(jnp.bfloat16))
    b2 = bias.astype(jnp.float32).reshape(1, Cout)

    taps = [(dy, dx) for dy in (-1, 0, 1) for dx in (-1, 0, 1)]
    body = functools.partial(_conv_body, N=N, H=H, W=W, rc=rc, taps=taps)

    out = pl.pallas_call(
        body,
        out_shape=jax.ShapeDtypeStruct((N, HW, Cout), jnp.float32),
        grid=(N,),
        in_specs=[
            pl.BlockSpec(memory_space=pl.ANY),
            pl.BlockSpec((kh * kw * Cin, Cout), lambda n: (0, 0)),
            pl.BlockSpec((1, Cout), lambda n: (0, 0)),
        ],
        out_specs=pl.BlockSpec((1, HW, Cout), lambda n: (n, 0, 0)),
        scratch_shapes=[pltpu.VMEM((2, H, W, Cin), x.dtype),
                        pltpu.VMEM((H, W, kh * kw * Cin), jnp.bfloat16),
                        pltpu.SemaphoreType.DMA((2,))],
        compiler_params=pltpu.CompilerParams(
            dimension_semantics=("arbitrary",),
            vmem_limit_bytes=48 * 1024 * 1024),
    )(xh, wt, b2)

    return jnp.transpose(out.reshape(N, H, W, Cout), (0, 3, 1, 2))
